# Initial kernel scaffold; baseline (speedup 1.0000x reference)
import jax
import jax.numpy as jnp
from jax.experimental import pallas as pl


def kernel(e1s, e2s, e3s, r1s, r2s, node_embedding, node_type, node_neighbors, rel_neighbors, node_weight, rel_weight, type_weight, rel_eye):
    raise NotImplementedError("write your pallas kernel here")


# speedup vs baseline: 8.9955x; 8.9955x over previous
import functools

import jax
import jax.numpy as jnp
from jax import lax
from jax.experimental import pallas as pl
from jax.experimental.pallas import tpu as pltpu
from jax.experimental.pallas import tpu_sc as plsc

_NODE = 100000
_NNUM = 32
_DIM = 128
_RELP1 = 65
_TYPEN = 16
_B = 8192
_AUGD = 144

_NC = 2
_NS = 16
_NW = _NC * _NS

_B_PER_W1 = _B // _NW


def _sc_query_gather(nn_hbm, rn_hbm, emb_hbm, e2s_hbm, e3s_hbm,
                     nbr_out, rel_out, e2e_out, e3e_out,
                     idx_v, nbr_v, rel_v, row_v, sem):
    wid = lax.axis_index("s") * _NC + lax.axis_index("c")
    base = wid * _B_PER_W1
    sl = pl.ds(base, _B_PER_W1)
    pltpu.sync_copy(e2s_hbm.at[sl], idx_v)
    pltpu.async_copy(nn_hbm.at[idx_v], nbr_v, sem).wait()
    pltpu.sync_copy(nbr_v, nbr_out.at[sl])
    pltpu.async_copy(rn_hbm.at[idx_v], rel_v, sem).wait()
    pltpu.sync_copy(rel_v, rel_out.at[sl])
    pltpu.async_copy(emb_hbm.at[idx_v], row_v, sem).wait()
    pltpu.sync_copy(row_v, e2e_out.at[sl])
    pltpu.sync_copy(e3s_hbm.at[sl], idx_v)
    pltpu.async_copy(emb_hbm.at[idx_v], row_v, sem).wait()
    pltpu.sync_copy(row_v, e3e_out.at[sl])


_BN = _B * _NNUM
_B_PER_W2 = _BN // _NW
_CH2 = 512
_NIT2 = _B_PER_W2 // _CH2


def _sc_nbr_gather(aug_hbm, idx_hbm, out_hbm, idx_v, rows_v, sem):
    wid = lax.axis_index("s") * _NC + lax.axis_index("c")
    base = wid * _B_PER_W2

    def body(i, carry):
        off = base + i * _CH2
        sl = pl.ds(off, _CH2)
        pltpu.sync_copy(idx_hbm.at[sl], idx_v)
        pltpu.async_copy(aug_hbm.at[idx_v], rows_v, sem).wait()
        pltpu.sync_copy(rows_v, out_hbm.at[sl])
        return carry

    lax.fori_loop(0, _NIT2, body, 0)


_BQ = 128
_NBLK = _B // _BQ


def _tc_compute(aug_ref, e2e_ref, e3e_ref, rel_ref, r1s_ref,
                relw_ref, typew_ref, out_ref, rel2_s):
    i = pl.program_id(0)

    @pl.when(i == 0)
    def _init():
        r1 = r1s_ref[...]
        for k in range(_RELP1):
            cnt = jnp.sum(jnp.where(r1 == k, 1.0, 0.0))
            rel2_s[k] = relw_ref[k] * (1.0 + cnt)
        out_ref[0, 0] = 0.0

    an = aug_ref[...]
    emb_n = an[:, :, :_DIM]
    nw = an[:, :, _DIM]
    ntf = an[:, :, _DIM + 1]
    cur = e2e_ref[...]
    e3 = e3e_ref[...]
    rel_ids = rel_ref[...]

    ew = jnp.zeros((_BQ, _NNUM), jnp.float32)
    for k in range(_RELP1):
        ew = ew + jnp.where(rel_ids == k, rel2_s[k], 0.0)
    tw = jnp.zeros((_BQ, _NNUM), jnp.float32)
    for t in range(_TYPEN):
        tw = tw + jnp.where(ntf == float(t), typew_ref[t], 0.0)

    w = nw + ew + tw
    w = w - jnp.max(w, axis=1, keepdims=True)
    w = jnp.exp(w)
    w = w / jnp.sum(w, axis=1, keepdims=True)

    d0 = jnp.sqrt(jnp.sum((cur - e3) ** 2, axis=1) + 1e-12)
    dn = jnp.sqrt(jnp.sum((emb_n - e3[:, None, :]) ** 2, axis=2) + 1e-12)
    sg = jnp.sign(d0[:, None] - dn)
    coef = w * sg
    moves = jnp.sum((emb_n - cur[:, None, :]) * coef[:, :, None], axis=1)
    new = cur + moves
    loss = jnp.sqrt(jnp.sum((new - e3) ** 2, axis=1) + 1e-12)
    out_ref[0, 0] += jnp.sum(loss)


def kernel(e1s, e2s, e3s, r1s, r2s, node_embedding, node_type,
           node_neighbors, rel_neighbors, node_weight, rel_weight,
           type_weight, rel_eye):
    e2s = e2s.astype(jnp.int32)
    e3s = e3s.astype(jnp.int32)
    nn = node_neighbors.astype(jnp.int32)
    rn = rel_neighbors.astype(jnp.int32)

    ntp = jnp.concatenate([node_type, node_type[-1:]]).astype(jnp.float32)
    aug = jnp.concatenate(
        [node_embedding,
         node_weight[:, None],
         ntp[:, None],
         jnp.zeros((_NODE + 1, _AUGD - _DIM - 2), jnp.float32)], axis=1)

    mesh = plsc.VectorSubcoreMesh(core_axis_name="c", subcore_axis_name="s")
    sc_params = pltpu.CompilerParams(use_tc_tiling_on_sc=False)

    sc1 = functools.partial(
        pl.kernel, mesh=mesh, compiler_params=sc_params,
        out_type=[
            jax.ShapeDtypeStruct((_B, _NNUM), jnp.int32),
            jax.ShapeDtypeStruct((_B, _NNUM), jnp.int32),
            jax.ShapeDtypeStruct((_B, _DIM), jnp.float32),
            jax.ShapeDtypeStruct((_B, _DIM), jnp.float32),
        ],
        scratch_types=[
            pltpu.VMEM((_B_PER_W1,), jnp.int32),
            pltpu.VMEM((_B_PER_W1, _NNUM), jnp.int32),
            pltpu.VMEM((_B_PER_W1, _NNUM), jnp.int32),
            pltpu.VMEM((_B_PER_W1, _DIM), jnp.float32),
            pltpu.SemaphoreType.DMA,
        ],
    )(_sc_query_gather)
    nbr_ids, rel_ids, emb_e2, emb_e3 = sc1(nn, rn, node_embedding, e2s, e3s)

    sc2 = functools.partial(
        pl.kernel, mesh=mesh, compiler_params=sc_params,
        out_type=jax.ShapeDtypeStruct((_BN, _AUGD), jnp.float32),
        scratch_types=[
            pltpu.VMEM((_CH2,), jnp.int32),
            pltpu.VMEM((_CH2, _AUGD), jnp.float32),
            pltpu.SemaphoreType.DMA,
        ],
    )(_sc_nbr_gather)
    aug_nbr = sc2(aug, nbr_ids.reshape(-1))

    out = pl.pallas_call(
        _tc_compute,
        grid=(_NBLK,),
        in_specs=[
            pl.BlockSpec((_BQ, _NNUM, _AUGD), lambda i: (i, 0, 0)),
            pl.BlockSpec((_BQ, _DIM), lambda i: (i, 0)),
            pl.BlockSpec((_BQ, _DIM), lambda i: (i, 0)),
            pl.BlockSpec((_BQ, _NNUM), lambda i: (i, 0)),
            pl.BlockSpec((_B // _DIM, _DIM), lambda i: (0, 0)),
            pl.BlockSpec(memory_space=pltpu.SMEM),
            pl.BlockSpec(memory_space=pltpu.SMEM),
        ],
        out_specs=pl.BlockSpec((1, 1), lambda i: (0, 0),
                               memory_space=pltpu.SMEM),
        out_shape=jax.ShapeDtypeStruct((1, 1), jnp.float32),
        scratch_shapes=[pltpu.SMEM((_RELP1,), jnp.float32)],
    )(aug_nbr.reshape(_B, _NNUM, _AUGD),
      emb_e2, emb_e3, rel_ids,
      r1s.astype(jnp.int32).reshape(_B // _DIM, _DIM),
      rel_weight, type_weight)

    return out[0, 0] / _B
